# baseline (speedup 1.0000x reference)
import functools

import jax
import jax.numpy as jnp
from jax import lax
from jax.experimental import pallas as pl


def _pick_block(n, target):
    best = None
    for b in range(8, target + 1, 8):
        if n % b == 0:
            best = b
    if best is None:
        raise ValueError(f"no block for n={n}")
    return best


def _mm_body(x_ref, w_ref, o_ref):
    r = jnp.dot(x_ref[...], w_ref[...], preferred_element_type=jnp.float32)
    o_ref[...] = r.astype(o_ref.dtype)


def _matmul(x, w, block, out_dtype=jnp.float32):
    n, k = x.shape
    f = w.shape[1]
    return pl.pallas_call(
        _mm_body,
        grid=(n // block,),
        in_specs=[
            pl.BlockSpec((block, k), lambda i: (i, 0)),
            pl.BlockSpec((k, f), lambda i: (0, 0)),
        ],
        out_specs=pl.BlockSpec((block, f), lambda i: (i, 0)),
        out_shape=jax.ShapeDtypeStruct((n, f), out_dtype),
    )(x, w)


def _agg_body_first(adj_ref, sup_ref, w_ref, abf_ref, o_ref):
    a = adj_ref[...].astype(jnp.bfloat16)
    abf_ref[...] = a
    z = jnp.dot(a, sup_ref[...], preferred_element_type=jnp.float32)
    z = jnp.maximum(z, 0.0)
    r = jnp.dot(z, w_ref[...], preferred_element_type=jnp.float32)
    o_ref[...] = r.astype(o_ref.dtype)


def _agg_first(adj, sup, w_next, block):
    n = adj.shape[0]
    f = sup.shape[1]
    fn = w_next.shape[1]
    adj_bf, sup_next = pl.pallas_call(
        _agg_body_first,
        grid=(n // block,),
        in_specs=[
            pl.BlockSpec((block, n), lambda i: (i, 0)),
            pl.BlockSpec((n, f), lambda i: (0, 0)),
            pl.BlockSpec((f, fn), lambda i: (0, 0)),
        ],
        out_specs=[
            pl.BlockSpec((block, n), lambda i: (i, 0)),
            pl.BlockSpec((block, fn), lambda i: (i, 0)),
        ],
        out_shape=[
            jax.ShapeDtypeStruct((n, n), jnp.bfloat16),
            jax.ShapeDtypeStruct((n, fn), jnp.bfloat16),
        ],
    )(adj, sup, w_next)
    return adj_bf, sup_next


def _agg_body_sup(adj_ref, sup_ref, w_ref, o_ref, *, relu):
    z = jnp.dot(adj_ref[...], sup_ref[...], preferred_element_type=jnp.float32)
    if relu:
        z = jnp.maximum(z, 0.0)
    r = jnp.dot(z, w_ref[...], preferred_element_type=jnp.float32)
    o_ref[...] = r.astype(o_ref.dtype)


def _agg_body_z(adj_ref, sup_ref, z_ref, *, relu):
    z = jnp.dot(adj_ref[...], sup_ref[...], preferred_element_type=jnp.float32)
    if relu:
        z = jnp.maximum(z, 0.0)
    z_ref[...] = z.astype(z_ref.dtype)


def _agg_body_both(adj_ref, sup_ref, w_ref, z_ref, o_ref, *, relu):
    z = jnp.dot(adj_ref[...], sup_ref[...], preferred_element_type=jnp.float32)
    if relu:
        z = jnp.maximum(z, 0.0)
    z_ref[...] = z.astype(z_ref.dtype)
    r = jnp.dot(z, w_ref[...], preferred_element_type=jnp.float32)
    o_ref[...] = r.astype(o_ref.dtype)


def _agg_pass(adj, sup, w_next, relu, want_z, block, sup_dtype=jnp.float32):
    n = adj.shape[0]
    f = sup.shape[1]
    in_specs = [
        pl.BlockSpec((block, n), lambda i: (i, 0)),
        pl.BlockSpec((n, f), lambda i: (0, 0)),
    ]
    args = [adj, sup]
    out_specs = []
    out_shape = []
    if want_z:
        out_specs.append(pl.BlockSpec((block, f), lambda i: (i, 0)))
        out_shape.append(jax.ShapeDtypeStruct((n, f), jnp.float32))
    if w_next is not None:
        fn = w_next.shape[1]
        in_specs.append(pl.BlockSpec((f, fn), lambda i: (0, 0)))
        args.append(w_next)
        out_specs.append(pl.BlockSpec((block, fn), lambda i: (i, 0)))
        out_shape.append(jax.ShapeDtypeStruct((n, fn), sup_dtype))
    if want_z and w_next is not None:
        body = functools.partial(_agg_body_both, relu=relu)
    elif want_z:
        body = functools.partial(_agg_body_z, relu=relu)
    else:
        body = functools.partial(_agg_body_sup, relu=relu)
    out = pl.pallas_call(
        body,
        grid=(n // block,),
        in_specs=in_specs,
        out_specs=out_specs,
        out_shape=out_shape,
    )(*args)
    return out[0] if len(out) == 1 else out


_NT = (((1,), (1,)), ((), ()))


def _adjhat_body(zgi_ref, zhi_ref, zg_ref, zh_ref, o_ref):
    zgi = zgi_ref[...].astype(jnp.bfloat16)
    zg = zg_ref[...].astype(jnp.bfloat16)
    a = lax.dot_general(zgi, zg, _NT, preferred_element_type=jnp.float32)
    b = lax.dot_general(zhi_ref[...], zh_ref[...], _NT,
                        preferred_element_type=jnp.float32)
    o_ref[...] = 1.0 + 0.5 * (jnp.tanh(0.5 * a) + jnp.tanh(0.5 * b))


def _adjhat(z_gae, z_hat, block):
    n, fg = z_gae.shape
    fh = z_hat.shape[1]
    zh_bf = z_hat.astype(jnp.bfloat16)
    return pl.pallas_call(
        _adjhat_body,
        grid=(n // block,),
        in_specs=[
            pl.BlockSpec((block, fg), lambda i: (i, 0)),
            pl.BlockSpec((block, fh), lambda i: (i, 0)),
            pl.BlockSpec((n, fg), lambda i: (0, 0)),
            pl.BlockSpec((n, fh), lambda i: (0, 0)),
        ],
        out_specs=pl.BlockSpec((block, n), lambda i: (i, 0)),
        out_shape=jax.ShapeDtypeStruct((n, n), jnp.float32),
    )(z_gae, zh_bf, z_gae, zh_bf)


def kernel(x, adj, W1, W2, W3, W4, W5, W6, W7, W8):
    n = adj.shape[0]
    bf = jnp.bfloat16
    blk = _pick_block(n, 1000)
    sup1 = _matmul(x, W1, _pick_block(n, 1000), out_dtype=bf)
    adj_bf, sup2 = _agg_first(adj, sup1, W2, _pick_block(n, 400))
    sup3 = _agg_pass(adj_bf, sup2, W3, relu=True, want_z=False, block=blk, sup_dtype=bf)
    sup4 = _agg_pass(adj_bf, sup3, W4, relu=True, want_z=False, block=blk, sup_dtype=bf)
    z_gae, sup5 = _agg_pass(adj_bf, sup4, W5, relu=False, want_z=True, block=blk, sup_dtype=bf)
    sup6 = _agg_pass(adj_bf, sup5, W6, relu=True, want_z=False, block=blk, sup_dtype=bf)
    sup7 = _agg_pass(adj_bf, sup6, W7, relu=True, want_z=False, block=blk, sup_dtype=bf)
    sup8 = _agg_pass(adj_bf, sup7, W8, relu=True, want_z=False, block=blk, sup_dtype=bf)
    z_hat = _agg_pass(adj_bf, sup8, None, relu=True, want_z=True, block=blk)
    adj_hat = _adjhat(z_gae, z_hat, _pick_block(n, 400))
    return (z_gae, z_hat, adj_hat)
